# baseline (speedup 1.0000x reference)
import functools

import jax
import jax.numpy as jnp
from jax import lax
from jax.experimental import pallas as pl
from jax.experimental.pallas import tpu as pltpu
from jax.experimental.pallas import tpu_sc as plsc

_NC = 2
_NS = 16
_NW = _NC * _NS
_B = 8192
_BPW = _B // _NW
_K = 32
_NG = 5
_SR = 128
_L = 16

_mesh = plsc.VectorSubcoreMesh(core_axis_name="c", subcore_axis_name="s")


@functools.partial(
    pl.kernel,
    mesh=_mesh,
    out_type=[jax.ShapeDtypeStruct((_K, _B), jnp.float32)] * _NG,
    scratch_types=[
        pltpu.VMEM((_NG * _BPW,), jnp.int32),
        pltpu.VMEM((_BPW,), jnp.int32),
        pltpu.VMEM((_BPW,), jnp.int32),
        pltpu.VMEM((_BPW, _SR), jnp.float32),
        pltpu.VMEM((_K, _BPW), jnp.float32),
        pltpu.SemaphoreType.DMA,
        pltpu.SemaphoreType.DMA,
    ],
    compiler_params=pltpu.CompilerParams(needs_layout_passes=False),
)
def _gather5(hs, ls, ts, hcs, tcs, embS_E, embS_R,
             o_hs, o_ls, o_ts, o_hcs, o_tcs,
             idx_v, sup_v, colb_v, rows_v, stage_v, isem, gsem):
    wid = lax.axis_index("s") * _NC + lax.axis_index("c")
    base = wid * _BPW
    srcs = (hs, ls, ts, hcs, tcs)
    tables = (embS_E, embS_R, embS_E, embS_E, embS_E)
    outs = (o_hs, o_ls, o_ts, o_hcs, o_tcs)

    icopies = [
        pltpu.async_copy(
            srcs[g].at[pl.ds(base, _BPW)],
            idx_v.at[pl.ds(g * _BPW, _BPW)],
            isem,
        )
        for g in range(_NG)
    ]
    for c in icopies:
        c.wait()

    for g in range(_NG):
        def prep(j, _, g=g):
            r = idx_v[pl.ds(g * _BPW + j * _L, _L)]
            sup_v[pl.ds(j * _L, _L)] = lax.shift_right_logical(r, 2)
            colb_v[pl.ds(j * _L, _L)] = lax.shift_left(
                lax.bitwise_and(r, 3), 5)
            return ()

        lax.fori_loop(0, _BPW // _L, prep, (), unroll=False)

        fetch = [
            pltpu.async_copy(
                tables[g].at[sup_v.at[pl.ds(j * 128, 128)]],
                rows_v.at[pl.ds(j * 128, 128), :],
                gsem,
            )
            for j in range(_BPW // 128)
        ]
        for c in fetch:
            c.wait()

        lanes = lax.iota(jnp.int32, _L)

        def extract(grp, _, g=g):
            rowi = grp * _L + lanes
            cb = colb_v[pl.ds(grp * _L, _L)]
            for c in range(_K):
                vals = plsc.load_gather(rows_v, [rowi, cb + c])
                stage_v[c, pl.ds(grp * _L, _L)] = vals
            return ()

        lax.fori_loop(0, _BPW // _L, extract, (), unroll=False)

        pltpu.sync_copy(stage_v, outs[g].at[:, pl.ds(base, _BPW)])


def kernel(X, emb_E, emb_R):
    half = X.shape[0] // 2
    hs = X[:half, 0]
    ls = X[:half, 1]
    ts = X[:half, 2]
    hcs = X[half:, 0]
    tcs = X[half:, 2]
    embS_E = emb_E.reshape(-1, _SR)
    embS_R = emb_R.reshape(-1, _SR)
    outs = _gather5(hs, ls, ts, hcs, tcs, embS_E, embS_R)
    return tuple(o.T for o in outs)


# speedup vs baseline: 1.0248x; 1.0248x over previous
import functools

import jax
import jax.numpy as jnp
from jax import lax
from jax.experimental import pallas as pl
from jax.experimental.pallas import tpu as pltpu
from jax.experimental.pallas import tpu_sc as plsc

_NC = 2
_NS = 16
_NW = _NC * _NS
_B = 8192
_BPW = _B // _NW
_K = 32
_NIDX = 5
_IC = 128
_NCH = _BPW // _IC

_mesh = plsc.VectorSubcoreMesh(core_axis_name="c", subcore_axis_name="s")


@functools.partial(
    pl.kernel,
    mesh=_mesh,
    out_type=[jax.ShapeDtypeStruct((_B, _K), jnp.float32)] * _NIDX,
    scratch_types=[
        pltpu.VMEM((_NIDX * _NCH, _IC), jnp.int32),
        pltpu.VMEM((_NIDX, _BPW, _K), jnp.float32),
        pltpu.SemaphoreType.DMA,
        pltpu.SemaphoreType.DMA,
    ],
    compiler_params=pltpu.CompilerParams(use_tc_tiling_on_sc=False),
)
def _gather5(hs, ls, ts, hcs, tcs, emb_E, emb_R,
             o_hs, o_ls, o_ts, o_hcs, o_tcs,
             idx_v, rows_v, isem, gsem):
    wid = lax.axis_index("s") * _NC + lax.axis_index("c")
    base = wid * _BPW
    srcs = (hs, ls, ts, hcs, tcs)
    tables = (emb_E, emb_R, emb_E, emb_E, emb_E)
    outs = (o_hs, o_ls, o_ts, o_hcs, o_tcs)

    icopies = [
        pltpu.async_copy(
            srcs[i].at[wid],
            idx_v.at[pl.ds(i * _NCH, _NCH)],
            isem,
        )
        for i in range(_NIDX)
    ]
    for c in icopies:
        c.wait()

    gcopies = []
    for i in range(_NIDX):
        for j in range(_NCH):
            gcopies.append(pltpu.async_copy(
                tables[i].at[idx_v.at[i * _NCH + j]],
                rows_v.at[i, pl.ds(j * _IC, _IC)],
                gsem,
            ))
    for cpy in gcopies:
        cpy.wait()

    for i in range(_NIDX):
        pltpu.sync_copy(rows_v.at[i], outs[i].at[pl.ds(base, _BPW)])


def kernel(X, emb_E, emb_R):
    half = X.shape[0] // 2
    hs = X[:half, 0].reshape(_NW, _NCH, _IC)
    ls = X[:half, 1].reshape(_NW, _NCH, _IC)
    ts = X[:half, 2].reshape(_NW, _NCH, _IC)
    hcs = X[half:, 0].reshape(_NW, _NCH, _IC)
    tcs = X[half:, 2].reshape(_NW, _NCH, _IC)
    return tuple(_gather5(hs, ls, ts, hcs, tcs, emb_E, emb_R))


# speedup vs baseline: 1.3533x; 1.3206x over previous
import functools

import jax
import jax.numpy as jnp
from jax import lax
from jax.experimental import pallas as pl
from jax.experimental.pallas import tpu as pltpu
from jax.experimental.pallas import tpu_sc as plsc

_NC = 2
_NS = 16
_NW = _NC * _NS
_B = 8192
_BPW = _B // _NW
_K = 32
_NG = 5
_L = 16
_CH = 64

_mesh = plsc.VectorSubcoreMesh(core_axis_name="c", subcore_axis_name="s")


@functools.partial(
    pl.kernel,
    mesh=_mesh,
    out_type=[jax.ShapeDtypeStruct((_K, _B), jnp.float32)] * _NG,
    scratch_types=[
        pltpu.VMEM((_NG * _BPW,), jnp.int32),
        pltpu.VMEM((_BPW,), jnp.int32),
        pltpu.VMEM((_CH * 8, _K), jnp.float32),
        pltpu.VMEM((_K, _BPW), jnp.float32),
        pltpu.SemaphoreType.DMA,
        pltpu.SemaphoreType.DMA,
    ],
    compiler_params=pltpu.CompilerParams(needs_layout_passes=False),
)
def _gather5(hs, ls, ts, hcs, tcs, emb_E, emb_R,
             o_hs, o_ls, o_ts, o_hcs, o_tcs,
             idx_v, rem_v, blk_v, stage_v, isem, gsem):
    wid = lax.axis_index("s") * _NC + lax.axis_index("c")
    base = wid * _BPW
    srcs = (hs, ls, ts, hcs, tcs)
    tables = (emb_E, emb_R, emb_E, emb_E, emb_E)
    outs = (o_hs, o_ls, o_ts, o_hcs, o_tcs)

    icopies = [
        pltpu.async_copy(
            srcs[g].at[pl.ds(base, _BPW)],
            idx_v.at[pl.ds(g * _BPW, _BPW)],
            isem,
        )
        for g in range(_NG)
    ]
    for c in icopies:
        c.wait()

    lanes = lax.iota(jnp.int32, _L)

    for g in range(_NG):
        tbl = tables[g]

        def chunk(c4, _, g=g, tbl=tbl):
            co = c4 * _CH

            def fetch(i, _):
                off = g * _BPW + co + i * _L
                v = idx_v[pl.ds(off, _L)]
                rem_v[pl.ds(co + i * _L, _L)] = lax.bitwise_and(v, 7)
                for k in range(_L):
                    b = lax.shift_left(
                        lax.shift_right_logical(v[k], 3), 3)
                    pltpu.async_copy(
                        tbl.at[pl.ds(pl.multiple_of(b, 8), 8), :],
                        blk_v.at[pl.ds((i * _L + k) * 8, 8), :],
                        gsem,
                    )
                return ()

            lax.fori_loop(0, _CH // _L, fetch, (), unroll=False)
            pltpu.make_async_copy(
                tbl.at[pl.ds(0, _CH * 8), :], blk_v, gsem
            ).wait()

            def extract(grp, _):
                rows = (grp * _L + lanes) * 8 + rem_v[
                    pl.ds(co + grp * _L, _L)]
                for c in range(_K):
                    vals = plsc.load_gather(
                        blk_v, [rows, jnp.full((_L,), c, jnp.int32)])
                    stage_v[c, pl.ds(co + grp * _L, _L)] = vals
                return ()

            lax.fori_loop(0, _CH // _L, extract, (), unroll=False)
            return ()

        lax.fori_loop(0, _BPW // _CH, chunk, (), unroll=False)

        pltpu.sync_copy(stage_v, outs[g].at[:, pl.ds(base, _BPW)])


def kernel(X, emb_E, emb_R):
    half = X.shape[0] // 2
    hs = X[:half, 0]
    ls = X[:half, 1]
    ts = X[:half, 2]
    hcs = X[half:, 0]
    tcs = X[half:, 2]
    outs = _gather5(hs, ls, ts, hcs, tcs, emb_E, emb_R)
    return tuple(o.T for o in outs)


# speedup vs baseline: 2.0694x; 1.5291x over previous
import functools

import jax
import jax.numpy as jnp
from jax import lax
from jax.experimental import pallas as pl
from jax.experimental.pallas import tpu as pltpu
from jax.experimental.pallas import tpu_sc as plsc

_NC = 2
_NS = 16
_NW = _NC * _NS
_B = 8192
_BPW = _B // _NW
_K = 32
_NG = 4
_L = 16
_CH = 64

_mesh = plsc.VectorSubcoreMesh(core_axis_name="c", subcore_axis_name="s")
_params = pltpu.CompilerParams(needs_layout_passes=False)


@functools.partial(
    pl.kernel,
    mesh=_mesh,
    out_type=jax.ShapeDtypeStruct((_K, _B), jnp.float32),
    scratch_types=[
        pltpu.VMEM((_BPW,), jnp.int32),
        pltpu.VMEM((_L * _K, 128), jnp.float32),
        pltpu.VMEM((_K, _BPW), jnp.float32),
        pltpu.SemaphoreType.DMA,
        pltpu.SemaphoreType.DMA,
    ],
    compiler_params=_params,
)
def _gather_native(ls, embT_R, o_ls, idx_v, blk_v, stage_v, isem, gsem):
    wid = lax.axis_index("s") * _NC + lax.axis_index("c")
    base = wid * _BPW
    pltpu.async_copy(ls.at[pl.ds(base, _BPW)], idx_v, isem).wait()
    lanes = lax.iota(jnp.int32, _L)

    def chunk(i, _):
        v = idx_v[pl.ds(i * _L, _L)]
        rem = lax.bitwise_and(v, 127)
        cps = []
        for k in range(_L):
            col0 = lax.shift_left(lax.shift_right_logical(v[k], 7), 7)
            cps.append(pltpu.async_copy(
                embT_R.at[:, pl.ds(pl.multiple_of(col0, 128), 128)],
                blk_v.at[pl.ds(k * _K, _K), :],
                gsem,
            ))
        for c in cps:
            c.wait()
        for c in range(_K):
            vals = plsc.load_gather(blk_v, [lanes * _K + c, rem])
            stage_v[c, pl.ds(i * _L, _L)] = vals
        return ()

    lax.fori_loop(0, _BPW // _L, chunk, (), unroll=False)
    pltpu.sync_copy(stage_v, o_ls.at[:, pl.ds(base, _BPW)])


@functools.partial(
    pl.kernel,
    mesh=_mesh,
    out_type=[jax.ShapeDtypeStruct((_K, _B), jnp.float32)] * _NG,
    scratch_types=[
        pltpu.VMEM((_NG * _BPW,), jnp.int32),
        pltpu.VMEM((_BPW,), jnp.int32),
        pltpu.VMEM((_CH * 8, _K), jnp.float32),
        pltpu.VMEM((_K, _BPW), jnp.float32),
        pltpu.SemaphoreType.DMA,
        pltpu.SemaphoreType.DMA,
    ],
    compiler_params=_params,
)
def _gather4(hs, ts, hcs, tcs, emb_E,
             o_hs, o_ts, o_hcs, o_tcs,
             idx_v, rem_v, blk_v, stage_v, isem, gsem):
    wid = lax.axis_index("s") * _NC + lax.axis_index("c")
    base = wid * _BPW
    srcs = (hs, ts, hcs, tcs)
    outs = (o_hs, o_ts, o_hcs, o_tcs)

    icopies = [
        pltpu.async_copy(
            srcs[g].at[pl.ds(base, _BPW)],
            idx_v.at[pl.ds(g * _BPW, _BPW)],
            isem,
        )
        for g in range(_NG)
    ]
    for c in icopies:
        c.wait()

    lanes = lax.iota(jnp.int32, _L)

    for g in range(_NG):
        def chunk(c4, _, g=g):
            co = c4 * _CH

            def fetch(i, _):
                off = g * _BPW + co + i * _L
                v = idx_v[pl.ds(off, _L)]
                rem_v[pl.ds(co + i * _L, _L)] = lax.bitwise_and(v, 7)
                for k in range(_L):
                    b = lax.shift_left(
                        lax.shift_right_logical(v[k], 3), 3)
                    pltpu.async_copy(
                        emb_E.at[pl.ds(pl.multiple_of(b, 8), 8), :],
                        blk_v.at[pl.ds((i * _L + k) * 8, 8), :],
                        gsem,
                    )
                return ()

            lax.fori_loop(0, _CH // _L, fetch, (), unroll=False)
            pltpu.make_async_copy(
                emb_E.at[pl.ds(0, _CH * 8), :], blk_v, gsem
            ).wait()

            def extract(grp, _):
                rows = (grp * _L + lanes) * 8 + rem_v[
                    pl.ds(co + grp * _L, _L)]
                for c in range(_K):
                    vals = plsc.load_gather(
                        blk_v, [rows, jnp.full((_L,), c, jnp.int32)])
                    stage_v[c, pl.ds(co + grp * _L, _L)] = vals
                return ()

            lax.fori_loop(0, _CH // _L, extract, (), unroll=False)
            return ()

        lax.fori_loop(0, _BPW // _CH, chunk, (), unroll=False)

        pltpu.sync_copy(stage_v, outs[g].at[:, pl.ds(base, _BPW)])


def kernel(X, emb_E, emb_R):
    half = X.shape[0] // 2
    hs = X[:half, 0]
    ls = X[:half, 1]
    ts = X[:half, 2]
    hcs = X[half:, 0]
    tcs = X[half:, 2]
    o_ls = _gather_native(ls, emb_R.T)
    o_hs, o_ts, o_hcs, o_tcs = _gather4(hs, ts, hcs, tcs, emb_E)
    return (o_hs.T, o_ls.T, o_ts.T, o_hcs.T, o_tcs.T)


# speedup vs baseline: 2.8617x; 1.3829x over previous
import functools

import jax
import jax.numpy as jnp
from jax import lax
from jax.experimental import pallas as pl
from jax.experimental.pallas import tpu as pltpu
from jax.experimental.pallas import tpu_sc as plsc

_NC = 2
_NS = 16
_NW = _NC * _NS
_B = 8192
_BPW = _B // _NW
_K = 32
_NG = 4
_L = 16
_CH = 64

_mesh = plsc.VectorSubcoreMesh(core_axis_name="c", subcore_axis_name="s")
_params = pltpu.CompilerParams(needs_layout_passes=False)


@functools.partial(
    pl.kernel,
    mesh=_mesh,
    out_type=jax.ShapeDtypeStruct((_K, _B), jnp.float32),
    scratch_types=[
        pltpu.VMEM((_BPW,), jnp.int32),
        pltpu.VMEM((_L * _K, 128), jnp.float32),
        pltpu.VMEM((_K, _BPW), jnp.float32),
        pltpu.SemaphoreType.DMA,
        pltpu.SemaphoreType.DMA,
    ],
    compiler_params=_params,
)
def _gather_native(ls, embT_R, o_ls, idx_v, blk_v, stage_v, isem, gsem):
    wid = lax.axis_index("s") * _NC + lax.axis_index("c")
    base = wid * _BPW
    pltpu.async_copy(ls.at[pl.ds(base, _BPW)], idx_v, isem).wait()
    lanes = lax.iota(jnp.int32, _L)

    def chunk(i, _):
        v = idx_v[pl.ds(i * _L, _L)]
        rem = lax.bitwise_and(v, 127)
        cps = []
        for k in range(_L):
            col0 = lax.shift_left(lax.shift_right_logical(v[k], 7), 7)
            cps.append(pltpu.async_copy(
                embT_R.at[:, pl.ds(pl.multiple_of(col0, 128), 128)],
                blk_v.at[pl.ds(k * _K, _K), :],
                gsem,
            ))
        for c in cps:
            c.wait()
        for c in range(_K):
            vals = plsc.load_gather(blk_v, [lanes * _K + c, rem])
            stage_v[c, pl.ds(i * _L, _L)] = vals
        return ()

    lax.fori_loop(0, _BPW // _L, chunk, (), unroll=False)
    pltpu.sync_copy(stage_v, o_ls.at[:, pl.ds(base, _BPW)])


@functools.partial(
    pl.kernel,
    mesh=_mesh,
    out_type=[jax.ShapeDtypeStruct((_K, _B), jnp.float32)] * _NG,
    scratch_types=[
        pltpu.VMEM((_NG * _BPW,), jnp.int32),
        pltpu.VMEM((_BPW,), jnp.int32),
        pltpu.VMEM((_CH * 8, _K), jnp.float32),
        pltpu.VMEM((_K, _BPW), jnp.float32),
        pltpu.SemaphoreType.DMA,
        pltpu.SemaphoreType.DMA,
    ],
    compiler_params=_params,
)
def _gather4(hs, ts, hcs, tcs, emb_E,
             o_hs, o_ts, o_hcs, o_tcs,
             idx_v, rem_v, blk_v, stage_v, isem, gsem):
    wid = lax.axis_index("s") * _NC + lax.axis_index("c")
    base = wid * _BPW
    srcs = (hs, ts, hcs, tcs)
    outs = (o_hs, o_ts, o_hcs, o_tcs)

    icopies = [
        pltpu.async_copy(
            srcs[g].at[pl.ds(base, _BPW)],
            idx_v.at[pl.ds(g * _BPW, _BPW)],
            isem,
        )
        for g in range(_NG)
    ]
    for c in icopies:
        c.wait()

    lanes = lax.iota(jnp.int32, _L)

    for g in range(_NG):
        def chunk(c4, _, g=g):
            co = c4 * _CH

            def fetch(i, _):
                off = g * _BPW + co + i * _L
                v = idx_v[pl.ds(off, _L)]
                rem_v[pl.ds(co + i * _L, _L)] = lax.bitwise_and(v, 7)
                for k in range(_L):
                    b = lax.shift_left(
                        lax.shift_right_logical(v[k], 3), 3)
                    pltpu.async_copy(
                        emb_E.at[0, pl.ds(pl.multiple_of(b, 8), 8), :],
                        blk_v.at[pl.ds((i * _L + k) * 8, 8), :],
                        gsem,
                    )
                return ()

            lax.fori_loop(0, _CH // _L, fetch, (), unroll=False)
            pltpu.make_async_copy(
                emb_E.at[0, pl.ds(0, _CH * 8), :], blk_v, gsem
            ).wait()

            def extract(grp, _):
                rows = (grp * _L + lanes) * 8 + rem_v[
                    pl.ds(co + grp * _L, _L)]
                for c in range(_K):
                    vals = plsc.load_gather(
                        blk_v, [rows, jnp.full((_L,), c, jnp.int32)])
                    stage_v[c, pl.ds(co + grp * _L, _L)] = vals
                return ()

            lax.fori_loop(0, _CH // _L, extract, (), unroll=False)
            return ()

        lax.fori_loop(0, _BPW // _CH, chunk, (), unroll=False)

        pltpu.sync_copy(stage_v, outs[g].at[:, pl.ds(base, _BPW)])


def kernel(X, emb_E, emb_R):
    half = X.shape[0] // 2
    hs = X[:half, 0]
    ls = X[:half, 1]
    ts = X[:half, 2]
    hcs = X[half:, 0]
    tcs = X[half:, 2]
    o_ls = _gather_native(ls, emb_R.T)
    o_hs, o_ts, o_hcs, o_tcs = _gather4(
        hs, ts, hcs, tcs, emb_E.reshape(1, -1, _K))
    return (o_hs.T, o_ls.T, o_ts.T, o_hcs.T, o_tcs.T)
